# baseline (speedup 1.0000x reference)
import functools

import jax
import jax.numpy as jnp
from jax import lax
from jax.experimental import pallas as pl
from jax.experimental.pallas import tpu as pltpu
from jax.experimental.pallas import tpu_sc as plsc

BATCH = 16384
DIM = 128
_NC = 2
_NS = 16
_NW = _NC * _NS
_ROWS = BATCH // 128
_R_PER_W = _ROWS // _NW


def _tc_matvec_body(f_ref, w_ref, b_ref, o_ref):
    f = f_ref[...]
    w = w_ref[...]
    acc = jnp.sum(f * w, axis=1, keepdims=True)
    o_ref[...] = acc + b_ref[0, 0]


def _tc_matvec(feature, fc_w, fc_b2):
    grid = (16,)
    blk = BATCH // grid[0]
    return pl.pallas_call(
        _tc_matvec_body,
        grid=grid,
        in_specs=[
            pl.BlockSpec((blk, DIM), lambda i: (i, 0)),
            pl.BlockSpec((1, DIM), lambda i: (0, 0)),
            pl.BlockSpec(memory_space=pltpu.SMEM),
        ],
        out_specs=pl.BlockSpec((blk, 1), lambda i: (i, 0)),
        out_shape=jax.ShapeDtypeStruct((BATCH, 1), jnp.float32),
    )(feature, fc_w, fc_b2)


def _sc_body(fc_hbm, uid_hbm, iid_hbm, bu_hbm, bi_hbm, out_hbm,
             uid_v, iid_v, bu_v, bi_v, fc_v, o_v, sem):
    wid = lax.axis_index("s") * _NC + lax.axis_index("c")
    r0 = wid * _R_PER_W
    pltpu.sync_copy(uid_hbm.at[pl.ds(r0, _R_PER_W)], uid_v)
    pltpu.sync_copy(iid_hbm.at[pl.ds(r0, _R_PER_W)], iid_v)
    copies = []
    for j in range(_R_PER_W):
        copies.append(pltpu.async_copy(bu_hbm.at[uid_v.at[j]], bu_v.at[j], sem))
        copies.append(pltpu.async_copy(bi_hbm.at[iid_v.at[j]], bi_v.at[j], sem))
    pltpu.sync_copy(fc_hbm.at[pl.ds(r0, _R_PER_W)], fc_v)
    for c in copies:
        c.wait()
    for j in range(_R_PER_W):
        for c in range(128 // 16):
            s = pl.ds(c * 16, 16)
            o_v[j, s] = fc_v[j, s] + bu_v[j, s] + bi_v[j, s]
    pltpu.sync_copy(o_v, out_hbm.at[pl.ds(r0, _R_PER_W)])


_sc_gather_add = functools.partial(
    pl.kernel,
    out_type=jax.ShapeDtypeStruct((_ROWS, 128), jnp.float32),
    mesh=plsc.VectorSubcoreMesh(core_axis_name="c", subcore_axis_name="s",
                                num_cores=_NC, num_subcores=_NS),
    scratch_types=[
        pltpu.VMEM((_R_PER_W, 128), jnp.int32),
        pltpu.VMEM((_R_PER_W, 128), jnp.int32),
        pltpu.VMEM((_R_PER_W, 128), jnp.float32),
        pltpu.VMEM((_R_PER_W, 128), jnp.float32),
        pltpu.VMEM((_R_PER_W, 128), jnp.float32),
        pltpu.VMEM((_R_PER_W, 128), jnp.float32),
        pltpu.SemaphoreType.DMA,
    ],
)(_sc_body)


def kernel(feature, user_id, item_id, fc_w, fc_b, b_users, b_items):
    fc = _tc_matvec(feature, fc_w, fc_b.reshape(1, 1))
    fc2 = fc.reshape(_ROWS, 128)
    uid2 = user_id.astype(jnp.int32).reshape(_ROWS, 128)
    iid2 = item_id.astype(jnp.int32).reshape(_ROWS, 128)
    bu_flat = b_users.reshape(-1)
    bi_flat = b_items.reshape(-1)
    out2 = _sc_gather_add(fc2, uid2, iid2, bu_flat, bi_flat)
    return out2.reshape(BATCH)


# speedup vs baseline: 1.5028x; 1.5028x over previous
import functools

import jax
import jax.numpy as jnp
from jax import lax
from jax.experimental import pallas as pl
from jax.experimental.pallas import tpu as pltpu
from jax.experimental.pallas import tpu_sc as plsc

BATCH = 16384
DIM = 128
TBL_PAD = 1048576
_NC = 2
_NS = 16
_NW = _NC * _NS
_B_PER_W = BATCH // _NW
_RG = _B_PER_W // 16
_IDR = _B_PER_W // 128


def _sc_body(f_hbm, uid_hbm, iid_hbm, bu_hbm, bi_hbm, w_hbm, out_hbm,
             f_v, uid_v, iid_v, bu_v, bi_v, w_v, o_v,
             sem_f, sem_ids, sem_w, sem_g):
    wid = lax.axis_index("s") * _NC + lax.axis_index("c")
    r0 = wid * _IDR

    cf = pltpu.async_copy(
        f_hbm.at[pl.ds(wid * _B_PER_W * DIM, _B_PER_W * DIM)], f_v, sem_f)
    cu = pltpu.async_copy(uid_hbm.at[pl.ds(r0, _IDR)], uid_v, sem_ids)
    ci = pltpu.async_copy(iid_hbm.at[pl.ds(r0, _IDR)], iid_v, sem_ids)
    cw = pltpu.async_copy(w_hbm, w_v, sem_w)

    cu.wait()
    ci.wait()

    gathers = []
    for j in range(_IDR):
        gathers.append(pltpu.async_copy(bu_hbm.at[uid_v.at[j]], bu_v.at[j], sem_g))
        gathers.append(pltpu.async_copy(bi_hbm.at[iid_v.at[j]], bi_v.at[j], sem_g))

    cw.wait()
    cf.wait()

    lane = jax.lax.iota(jnp.int32, 16)
    bias = w_v[pl.ds(DIM * 16, 16)]

    def mv_body(d, accs):
        wd = plsc.load_gather(w_v, [d * 16 + lane])
        out = []
        for g in range(_RG):
            idx = (lane * DIM + g * 16 * DIM) + d
            fv = plsc.load_gather(f_v, [idx])
            out.append(accs[g] + fv * wd)
        return tuple(out)

    accs = lax.fori_loop(0, DIM, mv_body, tuple(bias for _ in range(_RG)))
    for g in range(_RG):
        o_v[pl.ds(g * 16, 16)] = accs[g]

    for c in gathers:
        c.wait()

    for j in range(_IDR):
        for c in range(8):
            s16 = pl.ds(c * 16, 16)
            s = pl.ds(j * 128 + c * 16, 16)
            o_v[s] = o_v[s] + bu_v[j, s16] + bi_v[j, s16]

    pltpu.sync_copy(o_v, out_hbm.at[pl.ds(wid * _B_PER_W, _B_PER_W)])


_sc_lfm = functools.partial(
    pl.kernel,
    out_type=jax.ShapeDtypeStruct((BATCH,), jnp.float32),
    mesh=plsc.VectorSubcoreMesh(core_axis_name="c", subcore_axis_name="s",
                                num_cores=_NC, num_subcores=_NS),
    compiler_params=pltpu.CompilerParams(needs_layout_passes=False),
    scratch_types=[
        pltpu.VMEM((_B_PER_W * DIM,), jnp.float32),
        pltpu.VMEM((_IDR, 128), jnp.int32),
        pltpu.VMEM((_IDR, 128), jnp.int32),
        pltpu.VMEM((_IDR, 128), jnp.float32),
        pltpu.VMEM((_IDR, 128), jnp.float32),
        pltpu.VMEM((DIM * 16 + 16,), jnp.float32),
        pltpu.VMEM((_B_PER_W,), jnp.float32),
        pltpu.SemaphoreType.DMA,
        pltpu.SemaphoreType.DMA,
        pltpu.SemaphoreType.DMA,
        pltpu.SemaphoreType.DMA,
    ],
)(_sc_body)


def kernel(feature, user_id, item_id, fc_w, fc_b, b_users, b_items):
    f_flat = feature.reshape(-1)
    uid2 = user_id.reshape(BATCH // 128, 128)
    iid2 = item_id.reshape(BATCH // 128, 128)
    bu_flat = jnp.pad(b_users, ((0, TBL_PAD - b_users.shape[0]), (0, 0))).reshape(-1)
    bi_flat = jnp.pad(b_items, ((0, TBL_PAD - b_items.shape[0]), (0, 0))).reshape(-1)
    wtab = jnp.concatenate([
        jnp.repeat(fc_w[0], 16),
        jnp.repeat(fc_b, 16),
    ])
    return _sc_lfm(f_flat, uid2, iid2, bu_flat, bi_flat, wtab)
